# baseline (speedup 1.0000x reference)
import functools

import jax
import jax.numpy as jnp
from jax import lax
from jax.experimental import pallas as pl
from jax.experimental.pallas import tpu as pltpu
from jax.experimental.pallas import tpu_sc as plsc


_B = 4
_L = 4096
_D = 1024
_TC_B = 3
_SEQ_BLOCK = 2048
_NW = 32
_ROWS_PER_W = _L // 16
_CHUNK = 32


def _tc_add_body(x_ref, pos_ref, out_ref):
    out_ref[...] = x_ref[...] + pos_ref[...]


def _sc_body(x_hbm, pe_hbm, out_hbm, pe_v, xv0, xv1, isem0, isem1, osem0, osem1):
    wid = lax.axis_index("s")
    base = wid * _ROWS_PER_W

    n_chunks = _ROWS_PER_W // _CHUNK
    steps = [(ci, b) for ci in range(n_chunks) for b in range(_TC_B, _B)]
    bufs = [xv0, xv1]
    isems = [isem0, isem1]
    osems = [osem0, osem1]

    def in_row(ci, b):
        return b * _L + base + ci * _CHUNK

    def add_rows(buf):
        def row_body(r, c2):
            for c in range(0, _D, 16):
                buf[r, pl.ds(c, 16)] = buf[r, pl.ds(c, 16)] + pe_v[r, pl.ds(c, 16)]
            return c2

        lax.fori_loop(0, _CHUNK, row_body, 0)

    in_cms = [None, None]
    out_cms = [None, None]

    ci0, b0 = steps[0]
    in_cms[0] = pltpu.async_copy(
        x_hbm.at[pl.ds(in_row(ci0, b0), _CHUNK)], bufs[0], isems[0]
    )
    for i, (ci, b) in enumerate(steps):
        cur = i % 2
        nxt = (i + 1) % 2
        if b == _TC_B:
            pltpu.sync_copy(pe_hbm.at[pl.ds(base + ci * _CHUNK, _CHUNK)], pe_v)
        if i + 1 < len(steps):
            if out_cms[nxt] is not None:
                out_cms[nxt].wait()
                out_cms[nxt] = None
            ci1, b1 = steps[i + 1]
            in_cms[nxt] = pltpu.async_copy(
                x_hbm.at[pl.ds(in_row(ci1, b1), _CHUNK)], bufs[nxt], isems[nxt]
            )
        in_cms[cur].wait()
        add_rows(bufs[cur])
        out_cms[cur] = pltpu.async_copy(
            bufs[cur],
            out_hbm.at[pl.ds(base + ci * _CHUNK, _CHUNK)],
            osems[cur],
        )
    for cm in out_cms:
        if cm is not None:
            cm.wait()


def _sc_add(x2, pe):
    mesh = plsc.VectorSubcoreMesh(core_axis_name="c", subcore_axis_name="s", num_cores=1)
    kern = functools.partial(
        pl.kernel,
        mesh=mesh,
        out_type=jax.ShapeDtypeStruct((_L, _D), jnp.float32),
        scratch_types=[
            pltpu.VMEM((_CHUNK, _D), jnp.float32),
            pltpu.VMEM((_CHUNK, _D), jnp.float32),
            pltpu.VMEM((_CHUNK, _D), jnp.float32),
            pltpu.SemaphoreType.DMA,
            pltpu.SemaphoreType.DMA,
            pltpu.SemaphoreType.DMA,
            pltpu.SemaphoreType.DMA,
        ],
    )(_sc_body)
    return kern(x2, pe)


def _tc_add(x, pe):
    return pl.pallas_call(
        _tc_add_body,
        grid=(_L // _SEQ_BLOCK, _TC_B),
        in_specs=[
            pl.BlockSpec((1, _SEQ_BLOCK, _D), lambda s, b: (b, s, 0)),
            pl.BlockSpec((_SEQ_BLOCK, _D), lambda s, b: (s, 0)),
        ],
        out_specs=pl.BlockSpec((1, _SEQ_BLOCK, _D), lambda s, b: (b, s, 0)),
        out_shape=jax.ShapeDtypeStruct((_TC_B, _L, _D), jnp.float32),
    )(x, pe)


def kernel(x, pos_table):
    B, L, D = x.shape
    pe = pos_table[:L]
    out_sc = _sc_add(x.reshape(B * L, D), pe)
    out_tc = _tc_add(x, pe)
    return jnp.concatenate([out_tc, out_sc[None]], axis=0)


# speedup vs baseline: 2.6048x; 2.6048x over previous
import jax
import jax.numpy as jnp
from jax.experimental import pallas as pl
from jax.experimental.pallas import tpu as pltpu


_SEQ_BLOCK = 2048


def _add_kernel(x_ref, pos_ref, out_ref):
    out_ref[...] = x_ref[...] + pos_ref[...]


def kernel(x, pos_table):
    B, L, D = x.shape
    pe = pos_table[:L]
    grid = (L // _SEQ_BLOCK, B)
    return pl.pallas_call(
        _add_kernel,
        grid=grid,
        in_specs=[
            pl.BlockSpec((1, _SEQ_BLOCK, D), lambda s, b: (b, s, 0)),
            pl.BlockSpec((_SEQ_BLOCK, D), lambda s, b: (s, 0)),
        ],
        out_specs=pl.BlockSpec((1, _SEQ_BLOCK, D), lambda s, b: (b, s, 0)),
        out_shape=jax.ShapeDtypeStruct((B, L, D), x.dtype),
    )(x, pe)


# speedup vs baseline: 2.6351x; 1.0117x over previous
import jax
import jax.numpy as jnp
from jax.experimental import pallas as pl
from jax.experimental.pallas import tpu as pltpu


_SEQ_BLOCK = 2048


def _add_kernel(x_ref, pos_ref, out_ref):
    out_ref[...] = x_ref[...] + pos_ref[...]


def kernel(x, pos_table):
    B, L, D = x.shape
    pe = pos_table[:L]
    grid = (D // 512, B)
    return pl.pallas_call(
        _add_kernel,
        grid=grid,
        in_specs=[
            pl.BlockSpec((1, L, 512), lambda d, b: (b, 0, d)),
            pl.BlockSpec((L, 512), lambda d, b: (0, d)),
        ],
        out_specs=pl.BlockSpec((1, L, 512), lambda d, b: (b, 0, d)),
        out_shape=jax.ShapeDtypeStruct((B, L, D), x.dtype),
    )(x, pe)
